# Initial kernel scaffold; baseline (speedup 1.0000x reference)
import jax
import jax.numpy as jnp
from jax.experimental import pallas as pl


def kernel(class_embed_weight, indices):
    raise NotImplementedError("write your pallas kernel here")


# speedup vs baseline: 2.9690x; 2.9690x over previous
import functools

import jax
import jax.numpy as jnp
from jax import lax
from jax.experimental import pallas as pl
from jax.experimental.pallas import tpu as pltpu
from jax.experimental.pallas import tpu_sc as plsc

_NC = 2
_NS = 16
_NW = _NC * _NS

_B = 4096 * 50
_D = 128
_BPW = _B // _NW
_CHUNK = 128
_NCHUNK = _BPW // _CHUNK


@functools.partial(
    pl.kernel,
    out_type=jax.ShapeDtypeStruct((_B, _D), jnp.float32),
    mesh=plsc.VectorSubcoreMesh(
        core_axis_name="c", subcore_axis_name="s",
        num_cores=_NC, num_subcores=_NS),
    scratch_types=[
        pltpu.VMEM((_BPW,), jnp.int32),
        pltpu.VMEM((_CHUNK, _D), jnp.float32),
        pltpu.SemaphoreType.DMA,
    ],
)
def _gather_kernel(table_hbm, idx_hbm, out_hbm, idx_v, rows_v, sem):
    wid = lax.axis_index("s") * _NC + lax.axis_index("c")
    base = wid * _BPW
    pltpu.sync_copy(idx_hbm.at[pl.ds(base, _BPW)], idx_v)

    def step(j, carry):
        off = pl.multiple_of(j * _CHUNK, 8)
        pltpu.async_copy(table_hbm.at[idx_v.at[pl.ds(off, _CHUNK)]],
                         rows_v, sem).wait()
        pltpu.sync_copy(rows_v, out_hbm.at[pl.ds(base + off, _CHUNK)])
        return carry

    lax.fori_loop(0, _NCHUNK, step, 0)


def kernel(class_embed_weight, indices):
    idx_flat = indices.reshape(-1).astype(jnp.int32)
    out = _gather_kernel(class_embed_weight, idx_flat)
    return out.reshape(indices.shape + (_D,))


# speedup vs baseline: 3.3909x; 1.1421x over previous
import functools

import jax
import jax.numpy as jnp
from jax import lax
from jax.experimental import pallas as pl
from jax.experimental.pallas import tpu as pltpu
from jax.experimental.pallas import tpu_sc as plsc

_NC = 2
_NS = 16
_NW = _NC * _NS

_B = 4096 * 50
_D = 128
_BPW = _B // _NW
_CHUNK = 128
_NCHUNK = _BPW // _CHUNK
_NBUF = 4


@functools.partial(
    pl.kernel,
    out_type=jax.ShapeDtypeStruct((_B, _D), jnp.float32),
    mesh=plsc.VectorSubcoreMesh(
        core_axis_name="c", subcore_axis_name="s",
        num_cores=_NC, num_subcores=_NS),
    scratch_types=[
        pltpu.VMEM((_BPW,), jnp.int32),
        pltpu.VMEM((_NBUF, _CHUNK, _D), jnp.float32),
        pltpu.SemaphoreType.DMA,
        pltpu.SemaphoreType.DMA,
    ],
)
def _gather_kernel(table_hbm, idx_hbm, out_hbm, idx_v, rows_v, gsem, wsem):
    wid = lax.axis_index("s") * _NC + lax.axis_index("c")
    base = wid * _BPW
    pltpu.sync_copy(idx_hbm.at[pl.ds(base, _BPW)], idx_v)

    def g_desc(j, b):
        off = pl.multiple_of(j * _CHUNK, 8)
        return pltpu.make_async_copy(
            table_hbm.at[idx_v.at[pl.ds(off, _CHUNK)]], rows_v.at[b], gsem)

    def w_desc(j, b):
        off = pl.multiple_of(j * _CHUNK, 8)
        return pltpu.make_async_copy(
            rows_v.at[b], out_hbm.at[pl.ds(base + off, _CHUNK)], wsem)

    g_desc(0, 0).start()
    g_desc(1, 1).start()
    for jj in (0, 1):
        g_desc(jj, jj).wait()
        w_desc(jj, jj).start()
        g_desc(jj + 2, jj + 2).start()

    @pl.loop(4, _NCHUNK - 2, step=_NBUF)
    def _steady(j):
        for i in range(_NBUF):
            jj = j + i
            b = i
            bn = (i + 2) % _NBUF
            g_desc(jj, b).wait()
            w_desc(jj, b).start()
            w_desc(jj - 2, bn).wait()
            g_desc(jj + 2, bn).start()

    for jj, b in ((_NCHUNK - 2, 0), (_NCHUNK - 1, 1)):
        g_desc(jj, b).wait()
        w_desc(jj, b).start()
        w_desc(jj - 2, (b + 2) % _NBUF).wait()
    w_desc(_NCHUNK - 2, 0).wait()
    w_desc(_NCHUNK - 1, 1).wait()


def kernel(class_embed_weight, indices):
    idx_flat = indices.reshape(-1).astype(jnp.int32)
    out = _gather_kernel(class_embed_weight, idx_flat)
    return out.reshape(indices.shape + (_D,))
